# Initial kernel scaffold; baseline (speedup 1.0000x reference)
import jax
import jax.numpy as jnp
from jax.experimental import pallas as pl


def kernel(src, dst, edge_label_index, W1, b1, W2, b2):
    raise NotImplementedError("write your pallas kernel here")


# speedup vs baseline: 1.2888x; 1.2888x over previous
import functools

import jax
import jax.numpy as jnp
from jax import lax
from jax.experimental import pallas as pl
from jax.experimental.pallas import tpu as pltpu
from jax.experimental.pallas import tpu_sc as plsc

N_NODES = 10000
D_FEAT = 128
HIDDEN = 128
N_EDGES = 320000

NUM_CORES = 2
NUM_SUBCORES = 16
NUM_WORKERS = NUM_CORES * NUM_SUBCORES
E_PER_W = N_EDGES // NUM_WORKERS
CHUNK = 80
N_CHUNKS = E_PER_W // CHUNK
LANES = 16
VECS_PER_ROW = D_FEAT // LANES


def _sc_gather_mul(src, dst, row, col):
    mesh = plsc.VectorSubcoreMesh(core_axis_name="c", subcore_axis_name="s")

    @functools.partial(
        pl.kernel,
        mesh=mesh,
        out_type=jax.ShapeDtypeStruct((N_EDGES, D_FEAT), jnp.float32),
        scratch_types=[
            pltpu.VMEM((CHUNK,), jnp.int32),
            pltpu.VMEM((CHUNK,), jnp.int32),
            pltpu.VMEM((CHUNK, D_FEAT), jnp.float32),
            pltpu.VMEM((CHUNK, D_FEAT), jnp.float32),
            pltpu.VMEM((CHUNK, D_FEAT), jnp.float32),
            pltpu.SemaphoreType.DMA,
            pltpu.SemaphoreType.DMA,
        ],
    )
    def k(src_hbm, dst_hbm, row_hbm, col_hbm, z_hbm,
          row_v, col_v, s_v, d_v, z_v, sem_s, sem_d):
        wid = lax.axis_index("s") * NUM_CORES + lax.axis_index("c")
        wbase = wid * E_PER_W

        def body(kk, _):
            base = wbase + kk * CHUNK
            pltpu.sync_copy(row_hbm.at[pl.ds(base, CHUNK)], row_v)
            pltpu.sync_copy(col_hbm.at[pl.ds(base, CHUNK)], col_v)
            cs = pltpu.async_copy(src_hbm.at[row_v], s_v, sem_s)
            cd = pltpu.async_copy(dst_hbm.at[col_v], d_v, sem_d)
            cs.wait()
            cd.wait()

            def mul_body(i, _):
                for j in range(VECS_PER_ROW):
                    sl = pl.ds(j * LANES, LANES)
                    z_v[i, sl] = s_v[i, sl] * d_v[i, sl]
                return 0

            lax.fori_loop(0, CHUNK, mul_body, 0, unroll=2)
            pltpu.sync_copy(z_v, z_hbm.at[pl.ds(base, CHUNK)])
            return 0

        lax.fori_loop(0, N_CHUNKS, body, 0)

    return k(src, dst, row, col)


BLK_E = 512


def _mlp_body(z_ref, w1_ref, b1_ref, w2_ref, b2_ref, out_ref):
    h = jnp.dot(z_ref[...], w1_ref[...], preferred_element_type=jnp.float32)
    h = jnp.maximum(h + b1_ref[...], 0.0)
    out_ref[...] = (
        jnp.dot(h, w2_ref[...], preferred_element_type=jnp.float32) + b2_ref[...]
    )


def _tc_mlp(z, W1, b1, W2, b2):
    n_blk = N_EDGES // BLK_E
    return pl.pallas_call(
        _mlp_body,
        grid=(n_blk,),
        in_specs=[
            pl.BlockSpec((BLK_E, D_FEAT), lambda i: (i, 0)),
            pl.BlockSpec((D_FEAT, HIDDEN), lambda i: (0, 0)),
            pl.BlockSpec((1, HIDDEN), lambda i: (0, 0)),
            pl.BlockSpec((HIDDEN, 1), lambda i: (0, 0)),
            pl.BlockSpec((1, 1), lambda i: (0, 0)),
        ],
        out_specs=pl.BlockSpec((BLK_E, 1), lambda i: (i, 0)),
        out_shape=jax.ShapeDtypeStruct((N_EDGES, 1), jnp.float32),
    )(z, W1, b1, W2, b2)


def kernel(src, dst, edge_label_index, W1, b1, W2, b2):
    eli = edge_label_index.astype(jnp.int32)
    row, col = eli[0], eli[1]
    z = _sc_gather_mul(src, dst, row, col)
    out = _tc_mlp(z, W1, b1.reshape(1, HIDDEN), W2, b2.reshape(1, 1))
    return out.reshape(-1)


# speedup vs baseline: 1.7279x; 1.3408x over previous
import functools

import jax
import jax.numpy as jnp
from jax import lax
from jax.experimental import pallas as pl
from jax.experimental.pallas import tpu as pltpu
from jax.experimental.pallas import tpu_sc as plsc

N_NODES = 10000
D_FEAT = 128
HIDDEN = 128
N_EDGES = 320000

NUM_CORES = 2
NUM_SUBCORES = 16
NUM_WORKERS = NUM_CORES * NUM_SUBCORES
E_PER_W = N_EDGES // NUM_WORKERS
CHUNK = 40
NCH_W = E_PER_W // CHUNK
NBUF = 2
T_OUTER = NCH_W // NBUF
LANES = 16
VECS_PER_ROW = D_FEAT // LANES


def _sc_gather_mul(src, dst, row2d, col2d):
    mesh = plsc.VectorSubcoreMesh(core_axis_name="c", subcore_axis_name="s")

    @functools.partial(
        pl.kernel,
        mesh=mesh,
        out_type=jax.ShapeDtypeStruct((N_EDGES, D_FEAT), jnp.float32),
        scratch_types=[
            pltpu.VMEM((NCH_W, CHUNK), jnp.int32),
            pltpu.VMEM((NCH_W, CHUNK), jnp.int32),
            pltpu.VMEM((NBUF, CHUNK, D_FEAT), jnp.float32),
            pltpu.VMEM((NBUF, CHUNK, D_FEAT), jnp.float32),
            pltpu.VMEM((NBUF, CHUNK, D_FEAT), jnp.float32),
            pltpu.SemaphoreType.DMA,
            pltpu.SemaphoreType.DMA,
            pltpu.SemaphoreType.DMA,
            pltpu.SemaphoreType.DMA,
        ],
    )
    def k(src_hbm, dst_hbm, row_hbm, col_hbm, z_hbm,
          row_v, col_v, s_v, d_v, z_v, sg0, sg1, sw0, sw1):
        wid = lax.axis_index("s") * NUM_CORES + lax.axis_index("c")
        sems_g = (sg0, sg1)
        sems_w = (sw0, sw1)

        pltpu.sync_copy(row_hbm.at[wid], row_v)
        pltpu.sync_copy(col_hbm.at[wid], col_v)

        def issue_gather(ch, b):
            pltpu.async_copy(src_hbm.at[row_v.at[ch]], s_v.at[b], sems_g[b])
            pltpu.async_copy(dst_hbm.at[col_v.at[ch]], d_v.at[b], sems_g[b])

        def wait_gather(ch, b):
            pltpu.make_async_copy(
                src_hbm.at[row_v.at[ch]], s_v.at[b], sems_g[b]).wait()
            pltpu.make_async_copy(
                dst_hbm.at[col_v.at[ch]], d_v.at[b], sems_g[b]).wait()

        def out_slice(gch):
            return z_hbm.at[pl.ds(gch * CHUNK, CHUNK)]

        for b in range(NBUF):
            issue_gather(b, b)

        def body(t, _):
            for b in range(NBUF):
                ch = t * NBUF + b
                gch = wid * NCH_W + ch
                wait_gather(ch, b)

                @pl.when(t > 0)
                def _():
                    pltpu.make_async_copy(
                        z_v.at[b], out_slice(gch), sems_w[b]).wait()

                def mul_body(i, _):
                    for j in range(VECS_PER_ROW):
                        sl = pl.ds(j * LANES, LANES)
                        z_v[b, i, sl] = s_v[b, i, sl] * d_v[b, i, sl]
                    return 0

                lax.fori_loop(0, CHUNK, mul_body, 0, unroll=2)
                pltpu.async_copy(z_v.at[b], out_slice(gch), sems_w[b])

                @pl.when(t < T_OUTER - 1)
                def _():
                    issue_gather(ch + NBUF, b)
            return 0

        lax.fori_loop(0, T_OUTER, body, 0)

        for b in range(NBUF):
            gch = wid * NCH_W + (T_OUTER - 1) * NBUF + b
            pltpu.make_async_copy(z_v.at[b], out_slice(gch), sems_w[b]).wait()

    return k(src, dst, row2d, col2d)


BLK_E = 512


def _mlp_body(z_ref, w1_ref, b1_ref, w2_ref, b2_ref, out_ref):
    h = jnp.dot(z_ref[...], w1_ref[...], preferred_element_type=jnp.float32)
    h = jnp.maximum(h + b1_ref[...], 0.0)
    out_ref[...] = (
        jnp.dot(h, w2_ref[...], preferred_element_type=jnp.float32) + b2_ref[...]
    )


def _tc_mlp(z, W1, b1, W2, b2):
    n_blk = N_EDGES // BLK_E
    return pl.pallas_call(
        _mlp_body,
        grid=(n_blk,),
        in_specs=[
            pl.BlockSpec((BLK_E, D_FEAT), lambda i: (i, 0)),
            pl.BlockSpec((D_FEAT, HIDDEN), lambda i: (0, 0)),
            pl.BlockSpec((1, HIDDEN), lambda i: (0, 0)),
            pl.BlockSpec((HIDDEN, 1), lambda i: (0, 0)),
            pl.BlockSpec((1, 1), lambda i: (0, 0)),
        ],
        out_specs=pl.BlockSpec((BLK_E, 1), lambda i: (i, 0)),
        out_shape=jax.ShapeDtypeStruct((N_EDGES, 1), jnp.float32),
    )(z, W1, b1, W2, b2)


def kernel(src, dst, edge_label_index, W1, b1, W2, b2):
    eli = edge_label_index.astype(jnp.int32)
    row2d = eli[0].reshape(NUM_WORKERS, NCH_W, CHUNK)
    col2d = eli[1].reshape(NUM_WORKERS, NCH_W, CHUNK)
    z = _sc_gather_mul(src, dst, row2d, col2d)
    out = _tc_mlp(z, W1, b1.reshape(1, HIDDEN), W2, b2.reshape(1, 1))
    return out.reshape(-1)


# speedup vs baseline: 1.8885x; 1.0929x over previous
import functools

import jax
import jax.numpy as jnp
from jax import lax
from jax.experimental import pallas as pl
from jax.experimental.pallas import tpu as pltpu
from jax.experimental.pallas import tpu_sc as plsc

N_NODES = 10000
D_FEAT = 128
HIDDEN = 128
N_EDGES = 320000

NUM_CORES = 2
NUM_SUBCORES = 16
NUM_WORKERS = NUM_CORES * NUM_SUBCORES
E_PER_W = N_EDGES // NUM_WORKERS
CHUNK = 80
NCH_W = E_PER_W // CHUNK
NBUF = 2
T_OUTER = NCH_W // NBUF
R_EPI = NCH_W - T_OUTER * NBUF
LANES = 16
VECS_PER_ROW = D_FEAT // LANES


def _sc_gather_mul(src, dst, row3d, col3d):
    mesh = plsc.VectorSubcoreMesh(core_axis_name="c", subcore_axis_name="s")

    @functools.partial(
        pl.kernel,
        mesh=mesh,
        out_type=jax.ShapeDtypeStruct((N_EDGES, D_FEAT), jnp.float32),
        scratch_types=[
            pltpu.VMEM((NCH_W, CHUNK), jnp.int32),
            pltpu.VMEM((NCH_W, CHUNK), jnp.int32),
            pltpu.VMEM((NBUF, CHUNK, D_FEAT), jnp.float32),
            pltpu.VMEM((NBUF, CHUNK, D_FEAT), jnp.float32),
            pltpu.VMEM((NBUF, CHUNK, D_FEAT), jnp.float32),
        ] + [pltpu.SemaphoreType.DMA] * (2 * NBUF),
    )
    def k(src_hbm, dst_hbm, row_hbm, col_hbm, z_hbm,
          row_v, col_v, s_v, d_v, z_v, *sems):
        sems_g = sems[:NBUF]
        sems_w = sems[NBUF:]
        wid = lax.axis_index("s") * NUM_CORES + lax.axis_index("c")

        pltpu.sync_copy(row_hbm.at[wid], row_v)
        pltpu.sync_copy(col_hbm.at[wid], col_v)

        def issue_gather(ch, b):
            pltpu.async_copy(src_hbm.at[row_v.at[ch]], s_v.at[b], sems_g[b])
            pltpu.async_copy(dst_hbm.at[col_v.at[ch]], d_v.at[b], sems_g[b])

        def wait_gather(ch, b):
            pltpu.make_async_copy(
                src_hbm.at[row_v.at[ch]], s_v.at[b], sems_g[b]).wait()
            pltpu.make_async_copy(
                dst_hbm.at[col_v.at[ch]], d_v.at[b], sems_g[b]).wait()

        def out_slice(gch):
            return z_hbm.at[pl.ds(gch * CHUNK, CHUNK)]

        for b in range(NBUF):
            issue_gather(b, b)

        def multiply(b):
            def mul_body(i, _):
                for j in range(VECS_PER_ROW):
                    sl = pl.ds(j * LANES, LANES)
                    z_v[b, i, sl] = s_v[b, i, sl] * d_v[b, i, sl]
                return 0

            lax.fori_loop(0, CHUNK, mul_body, 0, unroll=2)

        def drain_write(b, gch):
            pltpu.make_async_copy(z_v.at[b], out_slice(gch), sems_w[b]).wait()

        def body(t, _):
            for b in range(NBUF):
                ch = t * NBUF + b
                gch = wid * NCH_W + ch
                wait_gather(ch, b)

                @pl.when(t > 0)
                def _():
                    drain_write(b, gch)

                multiply(b)
                pltpu.async_copy(z_v.at[b], out_slice(gch), sems_w[b])

                @pl.when(ch + NBUF < NCH_W)
                def _():
                    issue_gather(ch + NBUF, b)
            return 0

        lax.fori_loop(0, T_OUTER, body, 0)

        for r in range(R_EPI):
            ch = T_OUTER * NBUF + r
            b = ch % NBUF
            gch = wid * NCH_W + ch
            wait_gather(ch, b)
            drain_write(b, gch)
            multiply(b)
            pltpu.async_copy(z_v.at[b], out_slice(gch), sems_w[b])

        for b in range(NBUF):
            gch = wid * NCH_W + b
            drain_write(b, gch)

    return k(src, dst, row3d, col3d)


BLK_E = 512
N_BLK = N_EDGES // BLK_E


def _mlp_body(z_ref, w1_ref, b1_ref, w2_ref, b2_ref, out_ref):
    zb = z_ref[...].astype(jnp.bfloat16)
    h = jnp.dot(zb, w1_ref[...], preferred_element_type=jnp.float32)
    h = jnp.maximum(h + b1_ref[...], 0.0)
    res = (
        lax.dot_general(w2_ref[...], h, (((1,), (1,)), ((), ())),
                        preferred_element_type=jnp.float32)
        + b2_ref[...]
    )
    out_ref[...] = res.reshape(1, 1, BLK_E)


def _tc_mlp(z, W1, b1, w2r, b2):
    return pl.pallas_call(
        _mlp_body,
        grid=(N_BLK,),
        in_specs=[
            pl.BlockSpec((BLK_E, D_FEAT), lambda i: (i, 0)),
            pl.BlockSpec((D_FEAT, HIDDEN), lambda i: (0, 0)),
            pl.BlockSpec((1, HIDDEN), lambda i: (0, 0)),
            pl.BlockSpec((1, HIDDEN), lambda i: (0, 0)),
            pl.BlockSpec((1, 1), lambda i: (0, 0)),
        ],
        out_specs=pl.BlockSpec((1, 1, BLK_E), lambda i: (i, 0, 0)),
        out_shape=jax.ShapeDtypeStruct((N_BLK, 1, BLK_E), jnp.float32),
    )(z, W1, b1, w2r, b2)


def kernel(src, dst, edge_label_index, W1, b1, W2, b2):
    eli = edge_label_index.astype(jnp.int32)
    row3d = eli[0].reshape(NUM_WORKERS, NCH_W, CHUNK)
    col3d = eli[1].reshape(NUM_WORKERS, NCH_W, CHUNK)
    z = _sc_gather_mul(src, dst, row3d, col3d)
    out = _tc_mlp(
        z,
        W1.astype(jnp.bfloat16),
        b1.reshape(1, HIDDEN),
        W2.reshape(1, HIDDEN),
        b2.reshape(1, 1),
    )
    return out.reshape(-1)


# speedup vs baseline: 2.8409x; 1.5043x over previous
import functools

import jax
import jax.numpy as jnp
from jax import lax
from jax.experimental import pallas as pl
from jax.experimental.pallas import tpu as pltpu
from jax.experimental.pallas import tpu_sc as plsc

N_NODES = 10000
D_FEAT = 128
HIDDEN = 128
N_EDGES = 320000

NUM_CORES = 2
NUM_SUBCORES = 16
NUM_WORKERS = NUM_CORES * NUM_SUBCORES
E_PER_W = N_EDGES // NUM_WORKERS
CHUNK = 80
NCH_W = E_PER_W // CHUNK
NBUF = 2
T_OUTER = NCH_W // NBUF
R_EPI = NCH_W - T_OUTER * NBUF
LANES = 16
VECS_PER_ROW = D_FEAT // LANES


def _sc_gather_mul(src, dst, row3d, col3d):
    mesh = plsc.VectorSubcoreMesh(core_axis_name="c", subcore_axis_name="s")

    @functools.partial(
        pl.kernel,
        mesh=mesh,
        out_type=jax.ShapeDtypeStruct((N_EDGES, D_FEAT), jnp.float32),
        scratch_types=[
            pltpu.VMEM((NCH_W, CHUNK), jnp.int32),
            pltpu.VMEM((NCH_W, CHUNK), jnp.int32),
            pltpu.VMEM((NBUF, CHUNK, D_FEAT), jnp.float32),
            pltpu.VMEM((NBUF, CHUNK, D_FEAT), jnp.float32),
            pltpu.VMEM((NBUF, CHUNK, D_FEAT), jnp.float32),
        ] + [pltpu.SemaphoreType.DMA] * (2 * NBUF),
    )
    def k(src_hbm, dst_hbm, row_hbm, col_hbm, z_hbm,
          row_v, col_v, s_v, d_v, z_v, *sems):
        sems_g = sems[:NBUF]
        sems_w = sems[NBUF:]
        wid = lax.axis_index("s") * NUM_CORES + lax.axis_index("c")

        pltpu.sync_copy(row_hbm.at[wid], row_v)
        pltpu.sync_copy(col_hbm.at[wid], col_v)

        def issue_gather(ch, b):
            pltpu.async_copy(src_hbm.at[row_v.at[ch]], s_v.at[b], sems_g[b])
            pltpu.async_copy(dst_hbm.at[col_v.at[ch]], d_v.at[b], sems_g[b])

        def wait_gather(ch, b):
            pltpu.make_async_copy(
                src_hbm.at[row_v.at[ch]], s_v.at[b], sems_g[b]).wait()
            pltpu.make_async_copy(
                dst_hbm.at[col_v.at[ch]], d_v.at[b], sems_g[b]).wait()

        def out_slice(gch):
            return z_hbm.at[pl.ds(gch * CHUNK, CHUNK)]

        for b in range(NBUF):
            issue_gather(b, b)

        def multiply(b):
            def mul_body(i, _):
                for j in range(VECS_PER_ROW):
                    sl = pl.ds(j * LANES, LANES)
                    z_v[b, i, sl] = s_v[b, i, sl] * d_v[b, i, sl]
                return 0

            lax.fori_loop(0, CHUNK, mul_body, 0, unroll=2)

        def drain_write(b, gch):
            pltpu.make_async_copy(z_v.at[b], out_slice(gch), sems_w[b]).wait()

        def body(t, _):
            for b in range(NBUF):
                ch = t * NBUF + b
                gch = wid * NCH_W + ch
                wait_gather(ch, b)

                @pl.when(t > 0)
                def _():
                    drain_write(b, gch)

                multiply(b)
                pltpu.async_copy(z_v.at[b], out_slice(gch), sems_w[b])

                @pl.when(ch + NBUF < NCH_W)
                def _():
                    issue_gather(ch + NBUF, b)
            return 0

        lax.fori_loop(0, T_OUTER, body, 0)

        for r in range(R_EPI):
            ch = T_OUTER * NBUF + r
            b = ch % NBUF
            gch = wid * NCH_W + ch
            wait_gather(ch, b)
            drain_write(b, gch)
            multiply(b)
            pltpu.async_copy(z_v.at[b], out_slice(gch), sems_w[b])

        for b in range(NBUF):
            gch = wid * NCH_W + b
            drain_write(b, gch)

    return k(src, dst, row3d, col3d)


BLK_E = 2560
N_BLK = N_EDGES // BLK_E


def _mlp_body(z_ref, w1_ref, b1_ref, w2_ref, b2_ref, out_ref):
    zb = z_ref[...].astype(jnp.bfloat16)
    h = jnp.dot(zb, w1_ref[...], preferred_element_type=jnp.float32)
    h = jnp.maximum(h + b1_ref[...], 0.0)
    res = (
        lax.dot_general(w2_ref[...], h, (((1,), (1,)), ((), ())),
                        preferred_element_type=jnp.float32)
        + b2_ref[...]
    )
    out_ref[...] = res.reshape(1, 1, BLK_E)


def _tc_mlp(z, W1, b1, w2r, b2):
    return pl.pallas_call(
        _mlp_body,
        grid=(N_BLK,),
        in_specs=[
            pl.BlockSpec((BLK_E, D_FEAT), lambda i: (i, 0)),
            pl.BlockSpec((D_FEAT, HIDDEN), lambda i: (0, 0)),
            pl.BlockSpec((1, HIDDEN), lambda i: (0, 0)),
            pl.BlockSpec((1, HIDDEN), lambda i: (0, 0)),
            pl.BlockSpec((1, 1), lambda i: (0, 0)),
        ],
        out_specs=pl.BlockSpec((1, 1, BLK_E), lambda i: (i, 0, 0)),
        out_shape=jax.ShapeDtypeStruct((N_BLK, 1, BLK_E), jnp.float32),
        compiler_params=pltpu.CompilerParams(
            dimension_semantics=("arbitrary",)),
    )(z, W1, b1, w2r, b2)


def kernel(src, dst, edge_label_index, W1, b1, W2, b2):
    eli = edge_label_index.astype(jnp.int32)
    row3d = eli[0].reshape(NUM_WORKERS, NCH_W, CHUNK)
    col3d = eli[1].reshape(NUM_WORKERS, NCH_W, CHUNK)
    z = _sc_gather_mul(src, dst, row3d, col3d)
    out = _tc_mlp(
        z,
        W1.astype(jnp.bfloat16),
        b1.reshape(1, HIDDEN),
        W2.reshape(1, HIDDEN),
        b2.reshape(1, 1),
    )
    return out.reshape(-1)


# speedup vs baseline: 3.1484x; 1.1082x over previous
import functools

import jax
import jax.numpy as jnp
from jax import lax
from jax.experimental import pallas as pl
from jax.experimental.pallas import tpu as pltpu
from jax.experimental.pallas import tpu_sc as plsc

N_NODES = 10000
D_FEAT = 128
HIDDEN = 128
N_EDGES = 320000

NUM_CORES = 2
NUM_SUBCORES = 16
NUM_WORKERS = NUM_CORES * NUM_SUBCORES
E_PER_W = N_EDGES // NUM_WORKERS
CHUNK = 40
NCH_W = E_PER_W // CHUNK
NBUF = 5
NZB = 2
ROUND = 10
T_OUTER = NCH_W // ROUND
LANES = 16
VECS_PER_ROW = D_FEAT // LANES


def _sc_gather_mul(src, dst, row3d, col3d):
    mesh = plsc.VectorSubcoreMesh(core_axis_name="c", subcore_axis_name="s")

    @functools.partial(
        pl.kernel,
        mesh=mesh,
        out_type=jax.ShapeDtypeStruct((N_EDGES, D_FEAT), jnp.float32),
        scratch_types=[
            pltpu.VMEM((NCH_W, CHUNK), jnp.int32),
            pltpu.VMEM((NCH_W, CHUNK), jnp.int32),
            pltpu.VMEM((NBUF, CHUNK, D_FEAT), jnp.float32),
            pltpu.VMEM((NBUF, CHUNK, D_FEAT), jnp.float32),
            pltpu.VMEM((NZB, CHUNK, D_FEAT), jnp.float32),
        ] + [pltpu.SemaphoreType.DMA] * (NBUF + NZB),
    )
    def k(src_hbm, dst_hbm, row_hbm, col_hbm, z_hbm,
          row_v, col_v, s_v, d_v, z_v, *sems):
        sems_g = sems[:NBUF]
        sems_w = sems[NBUF:]
        wid = lax.axis_index("s") * NUM_CORES + lax.axis_index("c")
        wbase = wid * NCH_W

        pltpu.sync_copy(row_hbm.at[wid], row_v)
        pltpu.sync_copy(col_hbm.at[wid], col_v)

        def issue_gather(ch, b):
            pltpu.async_copy(src_hbm.at[row_v.at[ch]], s_v.at[b], sems_g[b])
            pltpu.async_copy(dst_hbm.at[col_v.at[ch]], d_v.at[b], sems_g[b])

        def wait_gather(ch, b):
            pltpu.make_async_copy(
                src_hbm.at[row_v.at[ch]], s_v.at[b], sems_g[b]).wait()
            pltpu.make_async_copy(
                dst_hbm.at[col_v.at[ch]], d_v.at[b], sems_g[b]).wait()

        def out_slice(gch):
            return z_hbm.at[pl.ds(gch * CHUNK, CHUNK)]

        def multiply(b, w):
            def mul_body(i, _):
                for j in range(VECS_PER_ROW):
                    sl = pl.ds(j * LANES, LANES)
                    z_v[w, i, sl] = s_v[b, i, sl] * d_v[b, i, sl]
                return 0

            lax.fori_loop(0, CHUNK, mul_body, 0, unroll=2)

        def drain_write(w, gch):
            pltpu.make_async_copy(z_v.at[w], out_slice(gch), sems_w[w]).wait()

        for b in range(NBUF):
            issue_gather(b, b)

        def body(t, _):
            for u in range(ROUND):
                b = u % NBUF
                w = u % NZB
                ch = t * ROUND + u
                gch = wbase + ch
                wait_gather(ch, b)

                @pl.when(ch >= NZB)
                def _():
                    drain_write(w, gch)

                multiply(b, w)
                pltpu.async_copy(z_v.at[w], out_slice(gch), sems_w[w])

                @pl.when(ch + NBUF < NCH_W)
                def _():
                    issue_gather(ch + NBUF, b)
            return 0

        lax.fori_loop(0, T_OUTER, body, 0)

        for w in range(NZB):
            drain_write(w, wbase + w)

    return k(src, dst, row3d, col3d)


BLK_E = 2560
N_BLK = N_EDGES // BLK_E


def _mlp_body(z_ref, w1_ref, b1_ref, w2_ref, b2_ref, out_ref):
    zb = z_ref[...].astype(jnp.bfloat16)
    h = jnp.dot(zb, w1_ref[...], preferred_element_type=jnp.float32)
    h = jnp.maximum(h + b1_ref[...], 0.0)
    res = (
        lax.dot_general(w2_ref[...], h, (((1,), (1,)), ((), ())),
                        preferred_element_type=jnp.float32)
        + b2_ref[...]
    )
    out_ref[...] = res.reshape(1, 1, BLK_E)


def _tc_mlp(z, W1, b1, w2r, b2):
    return pl.pallas_call(
        _mlp_body,
        grid=(N_BLK,),
        in_specs=[
            pl.BlockSpec((BLK_E, D_FEAT), lambda i: (i, 0)),
            pl.BlockSpec((D_FEAT, HIDDEN), lambda i: (0, 0)),
            pl.BlockSpec((1, HIDDEN), lambda i: (0, 0)),
            pl.BlockSpec((1, HIDDEN), lambda i: (0, 0)),
            pl.BlockSpec((1, 1), lambda i: (0, 0)),
        ],
        out_specs=pl.BlockSpec((1, 1, BLK_E), lambda i: (i, 0, 0)),
        out_shape=jax.ShapeDtypeStruct((N_BLK, 1, BLK_E), jnp.float32),
        compiler_params=pltpu.CompilerParams(
            dimension_semantics=("arbitrary",)),
    )(z, W1, b1, w2r, b2)


def kernel(src, dst, edge_label_index, W1, b1, W2, b2):
    eli = edge_label_index.astype(jnp.int32)
    row3d = eli[0].reshape(NUM_WORKERS, NCH_W, CHUNK)
    col3d = eli[1].reshape(NUM_WORKERS, NCH_W, CHUNK)
    z = _sc_gather_mul(src, dst, row3d, col3d)
    out = _tc_mlp(
        z,
        W1.astype(jnp.bfloat16),
        b1.reshape(1, HIDDEN),
        W2.reshape(1, HIDDEN),
        b2.reshape(1, 1),
    )
    return out.reshape(-1)


# speedup vs baseline: 3.2111x; 1.0199x over previous
import functools

import jax
import jax.numpy as jnp
from jax import lax
from jax.experimental import pallas as pl
from jax.experimental.pallas import tpu as pltpu
from jax.experimental.pallas import tpu_sc as plsc

N_NODES = 10000
D_FEAT = 128
HIDDEN = 128
N_EDGES = 320000

NUM_CORES = 2
NUM_SUBCORES = 16
NUM_WORKERS = NUM_CORES * NUM_SUBCORES
E_PER_W = N_EDGES // NUM_WORKERS
CHUNK = 40
NCH_W = E_PER_W // CHUNK
NBUF = 5
NZB = 2
ROUND = 10
T_OUTER = NCH_W // ROUND
LANES = 16
VECS_PER_ROW = D_FEAT // LANES


def _sc_gather_mul(src, dst, eli4d):
    mesh = plsc.VectorSubcoreMesh(core_axis_name="c", subcore_axis_name="s")

    @functools.partial(
        pl.kernel,
        mesh=mesh,
        out_type=jax.ShapeDtypeStruct((N_EDGES, D_FEAT), jnp.float32),
        scratch_types=[
            pltpu.VMEM((NCH_W, CHUNK), jnp.int32),
            pltpu.VMEM((NCH_W, CHUNK), jnp.int32),
            pltpu.VMEM((NBUF, CHUNK, D_FEAT), jnp.float32),
            pltpu.VMEM((NBUF, CHUNK, D_FEAT), jnp.float32),
            pltpu.VMEM((NZB, CHUNK, D_FEAT), jnp.float32),
        ] + [pltpu.SemaphoreType.DMA] * (NBUF + NZB),
    )
    def k(src_hbm, dst_hbm, eli_hbm, z_hbm,
          row_v, col_v, s_v, d_v, z_v, *sems):
        sems_g = sems[:NBUF]
        sems_w = sems[NBUF:]
        wid = lax.axis_index("s") * NUM_CORES + lax.axis_index("c")
        wbase = wid * NCH_W

        pltpu.sync_copy(eli_hbm.at[0].at[wid], row_v)
        pltpu.sync_copy(eli_hbm.at[1].at[wid], col_v)

        def issue_gather(ch, b):
            pltpu.async_copy(src_hbm.at[row_v.at[ch]], s_v.at[b], sems_g[b])
            pltpu.async_copy(dst_hbm.at[col_v.at[ch]], d_v.at[b], sems_g[b])

        def wait_gather(ch, b):
            pltpu.make_async_copy(
                src_hbm.at[row_v.at[ch]], s_v.at[b], sems_g[b]).wait()
            pltpu.make_async_copy(
                dst_hbm.at[col_v.at[ch]], d_v.at[b], sems_g[b]).wait()

        def out_slice(gch):
            return z_hbm.at[pl.ds(gch * CHUNK, CHUNK)]

        def multiply(b, w):
            def mul_body(i, _):
                for j in range(VECS_PER_ROW):
                    sl = pl.ds(j * LANES, LANES)
                    z_v[w, i, sl] = s_v[b, i, sl] * d_v[b, i, sl]
                return 0

            lax.fori_loop(0, CHUNK, mul_body, 0, unroll=2)

        def drain_write(w, gch):
            pltpu.make_async_copy(z_v.at[w], out_slice(gch), sems_w[w]).wait()

        for b in range(NBUF):
            issue_gather(b, b)

        def body(t, _):
            for u in range(ROUND):
                b = u % NBUF
                w = u % NZB
                ch = t * ROUND + u
                gch = wbase + ch
                wait_gather(ch, b)

                @pl.when(ch >= NZB)
                def _():
                    drain_write(w, gch)

                multiply(b, w)
                pltpu.async_copy(z_v.at[w], out_slice(gch), sems_w[w])

                @pl.when(ch + NBUF < NCH_W)
                def _():
                    issue_gather(ch + NBUF, b)
            return 0

        lax.fori_loop(0, T_OUTER, body, 0)

        for w in range(NZB):
            drain_write(w, wbase + w)

    return k(src, dst, eli4d)


BLK_E = 2560
N_BLK = N_EDGES // BLK_E


def _mlp_body(z_ref, w1_ref, b1_ref, w2_ref, b2_ref, out_ref):
    zb = z_ref[...].astype(jnp.bfloat16)
    h = jnp.dot(zb, w1_ref[...], preferred_element_type=jnp.float32)
    h = jnp.maximum(h + b1_ref[...], 0.0)
    res = (
        lax.dot_general(w2_ref[...], h, (((1,), (1,)), ((), ())),
                        preferred_element_type=jnp.float32)
        + b2_ref[...]
    )
    out_ref[...] = res.reshape(1, 1, BLK_E)


def _tc_mlp(z, W1, b1, w2r, b2):
    return pl.pallas_call(
        _mlp_body,
        grid=(N_BLK,),
        in_specs=[
            pl.BlockSpec((BLK_E, D_FEAT), lambda i: (i, 0)),
            pl.BlockSpec((D_FEAT, HIDDEN), lambda i: (0, 0)),
            pl.BlockSpec((1, HIDDEN), lambda i: (0, 0)),
            pl.BlockSpec((1, HIDDEN), lambda i: (0, 0)),
            pl.BlockSpec((1, 1), lambda i: (0, 0)),
        ],
        out_specs=pl.BlockSpec((1, 1, BLK_E), lambda i: (i, 0, 0)),
        out_shape=jax.ShapeDtypeStruct((N_BLK, 1, BLK_E), jnp.float32),
        compiler_params=pltpu.CompilerParams(
            dimension_semantics=("arbitrary",)),
    )(z, W1, b1, w2r, b2)


def kernel(src, dst, edge_label_index, W1, b1, W2, b2):
    eli4d = edge_label_index.astype(jnp.int32).reshape(
        2, NUM_WORKERS, NCH_W, CHUNK)
    z = _sc_gather_mul(src, dst, eli4d)
    out = _tc_mlp(
        z,
        W1.astype(jnp.bfloat16),
        b1.reshape(1, HIDDEN),
        W2.reshape(1, HIDDEN),
        b2.reshape(1, 1),
    )
    return out.reshape(-1)


# speedup vs baseline: 3.5204x; 1.0963x over previous
import functools

import jax
import jax.numpy as jnp
from jax import lax
from jax.experimental import pallas as pl
from jax.experimental.pallas import tpu as pltpu
from jax.experimental.pallas import tpu_sc as plsc

N_NODES = 10000
D_FEAT = 128
HIDDEN = 128
N_EDGES = 320000

NUM_CORES = 2
NUM_SUBCORES = 16
NUM_WORKERS = NUM_CORES * NUM_SUBCORES
E_PER_W = N_EDGES // NUM_WORKERS
CHUNK = 40
NCH_W = E_PER_W // CHUNK
NBUF = 5
NZB = 2
ROUND = 10
T_OUTER = NCH_W // ROUND
LANES = 16
VECS_PER_ROW = D_FEAT // LANES


def _sc_gather_mul(src, dst, eli4d):
    mesh = plsc.VectorSubcoreMesh(core_axis_name="c", subcore_axis_name="s")

    @functools.partial(
        pl.kernel,
        mesh=mesh,
        out_type=jax.ShapeDtypeStruct((N_EDGES, D_FEAT), jnp.float32),
        scratch_types=[
            pltpu.VMEM((NCH_W, CHUNK), jnp.int32),
            pltpu.VMEM((NCH_W, CHUNK), jnp.int32),
            pltpu.VMEM((NBUF, CHUNK, D_FEAT), jnp.float32),
            pltpu.VMEM((NBUF, CHUNK, D_FEAT), jnp.float32),
            pltpu.VMEM((NZB, CHUNK, D_FEAT), jnp.float32),
        ] + [pltpu.SemaphoreType.DMA] * (NBUF + NZB),
    )
    def k(src_hbm, dst_hbm, eli_hbm, z_hbm,
          row_v, col_v, s_v, d_v, z_v, *sems):
        sems_g = sems[:NBUF]
        sems_w = sems[NBUF:]
        wid = lax.axis_index("s") * NUM_CORES + lax.axis_index("c")
        wbase = wid * NCH_W

        pltpu.sync_copy(eli_hbm.at[0].at[wid], row_v)
        pltpu.sync_copy(eli_hbm.at[1].at[wid], col_v)

        def issue_gather(ch, b):
            pltpu.async_copy(src_hbm.at[row_v.at[ch]], s_v.at[b], sems_g[b])
            pltpu.async_copy(dst_hbm.at[col_v.at[ch]], d_v.at[b], sems_g[b])

        def wait_gather(ch, b):
            pltpu.make_async_copy(
                src_hbm.at[row_v.at[ch]], s_v.at[b], sems_g[b]).wait()
            pltpu.make_async_copy(
                dst_hbm.at[col_v.at[ch]], d_v.at[b], sems_g[b]).wait()

        def out_slice(gch):
            return z_hbm.at[pl.ds(gch * CHUNK, CHUNK)]

        def multiply(b, w):
            def mul_body(i, _):
                for j in range(VECS_PER_ROW):
                    sl = pl.ds(j * LANES, LANES)
                    z_v[w, i, sl] = s_v[b, i, sl] * d_v[b, i, sl]
                return 0

            lax.fori_loop(0, CHUNK, mul_body, 0, unroll=2)

        def drain_write(w, gch):
            pltpu.make_async_copy(z_v.at[w], out_slice(gch), sems_w[w]).wait()

        for b in range(NBUF):
            issue_gather(b, b)

        def body(t, _):
            for u in range(ROUND):
                b = u % NBUF
                w = u % NZB
                ch = t * ROUND + u
                gch = wbase + ch
                wait_gather(ch, b)

                @pl.when(ch >= NZB)
                def _():
                    drain_write(w, gch)

                multiply(b, w)
                pltpu.async_copy(z_v.at[w], out_slice(gch), sems_w[w])

                @pl.when(ch + NBUF < NCH_W)
                def _():
                    issue_gather(ch + NBUF, b)
            return 0

        lax.fori_loop(0, T_OUTER, body, 0)

        for w in range(NZB):
            drain_write(w, wbase + w)

    return k(src, dst, eli4d)


BLK_E = 6400
N_BLK = N_EDGES // BLK_E


def _mlp_body(z_ref, w1_ref, b1_ref, w2_ref, b2_ref, out_ref):
    zb = z_ref[...].astype(jnp.bfloat16)
    h = jnp.dot(zb, w1_ref[...], preferred_element_type=jnp.float32)
    h = jnp.maximum(h + b1_ref[...], 0.0)
    res = (
        lax.dot_general(w2_ref[...], h, (((1,), (1,)), ((), ())),
                        preferred_element_type=jnp.float32)
        + b2_ref[...]
    )
    out_ref[...] = res.reshape(1, 1, BLK_E)


def _tc_mlp(z, W1, b1, w2r, b2):
    return pl.pallas_call(
        _mlp_body,
        grid=(N_BLK,),
        in_specs=[
            pl.BlockSpec((BLK_E, D_FEAT), lambda i: (i, 0)),
            pl.BlockSpec((D_FEAT, HIDDEN), lambda i: (0, 0)),
            pl.BlockSpec((1, HIDDEN), lambda i: (0, 0)),
            pl.BlockSpec((1, HIDDEN), lambda i: (0, 0)),
            pl.BlockSpec((1, 1), lambda i: (0, 0)),
        ],
        out_specs=pl.BlockSpec((1, 1, BLK_E), lambda i: (i, 0, 0)),
        out_shape=jax.ShapeDtypeStruct((N_BLK, 1, BLK_E), jnp.float32),
        compiler_params=pltpu.CompilerParams(
            dimension_semantics=("arbitrary",)),
    )(z, W1, b1, w2r, b2)


def kernel(src, dst, edge_label_index, W1, b1, W2, b2):
    eli4d = edge_label_index.astype(jnp.int32).reshape(
        2, NUM_WORKERS, NCH_W, CHUNK)
    z = _sc_gather_mul(src, dst, eli4d)
    out = _tc_mlp(
        z,
        W1.astype(jnp.bfloat16),
        b1.reshape(1, HIDDEN),
        W2.reshape(1, HIDDEN),
        b2.reshape(1, 1),
    )
    return out.reshape(-1)
